# baseline (speedup 1.0000x reference)
import functools

import jax
import jax.numpy as jnp
from jax import lax
from jax.experimental import pallas as pl
from jax.experimental.pallas import tpu as pltpu
from jax.experimental.pallas import tpu_sc as plsc

_N = 10000
_E = 320000
_D = 128

_NC = 2
_NS = 16
_L = 16

_EPT = _E // _NS
_CB = 128
_SUP = 4096
_NSUP = -(-_EPT // _SUP)
_TAIL = _EPT - (_EPT // _CB) * _CB
_RPT = (_N // _NS) // 8 * 8
_RREM = _N - _RPT * _NS

_ROW_BLK = 1000


def _dense_body(mean_ref, var_ref, wm_ref, bm_ref, wv_ref, bv_ref, out_ref):
    m = jnp.dot(mean_ref[...], wm_ref[...],
                preferred_element_type=jnp.float32) + bm_ref[...]
    m = jnp.where(m > 0.0, m, jnp.exp(m) - 1.0)
    v = jnp.dot(var_ref[...], wv_ref[...],
                preferred_element_type=jnp.float32) + bv_ref[...]
    v = jnp.maximum(v, 0.0) + 1e-6
    att = jnp.exp(-v)
    out_ref[0] = m * att
    out_ref[1] = (v * att) * att


def _dense_transform(mean, var, W_mean, b_mean, W_var, b_var):
    return pl.pallas_call(
        _dense_body,
        grid=(_N // _ROW_BLK,),
        in_specs=[
            pl.BlockSpec((_ROW_BLK, _D), lambda i: (i, 0)),
            pl.BlockSpec((_ROW_BLK, _D), lambda i: (i, 0)),
            pl.BlockSpec((_D, _D), lambda i: (0, 0)),
            pl.BlockSpec((1, _D), lambda i: (0, 0)),
            pl.BlockSpec((_D, _D), lambda i: (0, 0)),
            pl.BlockSpec((1, _D), lambda i: (0, 0)),
        ],
        out_specs=pl.BlockSpec((2, _ROW_BLK, _D), lambda i: (0, i, 0)),
        out_shape=jax.ShapeDtypeStruct((2, _N, _D), jnp.float32),
    )(mean, var, W_mean, b_mean.reshape(1, _D), W_var, b_var.reshape(1, _D))


def _sc_agg_body(x_hbm, dst_hbm, src_hbm, w_hbm, z_hbm, m_out, v_out,
                 srcs, dsts, ws, rows0, rows1, rowst, acc, sem0, sem1):
    sid = lax.axis_index("s")
    cid = lax.axis_index("c")

    pltpu.sync_copy(z_hbm.at[pl.ds(sid * _RPT, _RPT)],
                    acc.at[pl.ds(sid * _RPT, _RPT)])

    @pl.when(sid == 0)
    def _():
        pltpu.sync_copy(z_hbm.at[pl.ds(_NS * _RPT, _RREM)],
                        acc.at[pl.ds(_NS * _RPT, _RREM)])

    plsc.subcore_barrier()

    ebase = sid * _EPT
    xoff = cid * _N
    woff = cid * _E

    def issue(ci, buf, sem):
        pltpu.async_copy(x_hbm.at[srcs.at[pl.ds(ci * _CB, _CB)]], buf, sem)

    def drain(buf, sem):
        pltpu.make_async_copy(x_hbm.at[pl.ds(0, _CB)], buf, sem).wait()

    def process(ci, buf):
        base = ci * _CB

        @plsc.parallel_loop(0, _CB, unroll=8)
        def _(e):
            wv = plsc.load_gather(ws, [jnp.full((_L,), base + e, jnp.int32)])
            for j in range(_D // _L):
                sl = pl.ds(j * _L, _L)
                buf[e, sl] = buf[e, sl] * wv

        pltpu.sync_copy(buf, acc.at[dsts.at[pl.ds(base, _CB)]], add=True)

    for s in range(_NSUP):
        slen = _SUP if s < _NSUP - 1 else _EPT - (_NSUP - 1) * _SUP
        nfull = slen // _CB
        stail = slen - nfull * _CB
        sb = ebase + s * _SUP

        pltpu.sync_copy(src_hbm.at[pl.ds(sb, slen)], srcs.at[pl.ds(0, slen)])
        pltpu.sync_copy(dst_hbm.at[pl.ds(sb, slen)], dsts.at[pl.ds(0, slen)])
        pltpu.sync_copy(w_hbm.at[pl.ds(woff + sb, slen)],
                        ws.at[pl.ds(0, slen)])

        @plsc.parallel_loop(0, slen // _L, unroll=4)
        def _(i):
            sl = pl.ds(i * _L, _L)
            srcs[sl] = srcs[sl] + xoff

        issue(0, rows0, sem0)

        def ring_body(i, carry, nfull=nfull):
            g = i * 2
            issue(g + 1, rows1, sem1)
            drain(rows0, sem0)
            process(g, rows0)

            @pl.when(g + 2 < nfull)
            def _():
                issue(g + 2, rows0, sem0)

            drain(rows1, sem1)
            process(g + 1, rows1)
            return carry

        lax.fori_loop(0, nfull // 2, ring_body, 0)

        if stail:
            tb = nfull * _CB
            pltpu.async_copy(
                x_hbm.at[srcs.at[pl.ds(tb, stail)]], rowst, sem0).wait()

            @plsc.parallel_loop(0, stail, unroll=4)
            def _(e):
                wv = plsc.load_gather(
                    ws, [jnp.full((_L,), tb + e, jnp.int32)])
                for j in range(_D // _L):
                    sl = pl.ds(j * _L, _L)
                    rowst[e, sl] = rowst[e, sl] * wv
            pltpu.sync_copy(rowst, acc.at[dsts.at[pl.ds(tb, stail)]],
                            add=True)

    plsc.subcore_barrier()

    @pl.when(cid == 0)
    def _():
        pltpu.sync_copy(acc.at[pl.ds(sid * _RPT, _RPT)],
                        m_out.at[pl.ds(sid * _RPT, _RPT)])

    @pl.when((cid == 0) & (sid == 0))
    def _():
        pltpu.sync_copy(acc.at[pl.ds(_NS * _RPT, _RREM)],
                        m_out.at[pl.ds(_NS * _RPT, _RREM)])

    @pl.when(cid == 1)
    def _():
        pltpu.sync_copy(acc.at[pl.ds(sid * _RPT, _RPT)],
                        v_out.at[pl.ds(sid * _RPT, _RPT)])

    @pl.when((cid == 1) & (sid == 0))
    def _():
        pltpu.sync_copy(acc.at[pl.ds(_NS * _RPT, _RREM)],
                        v_out.at[pl.ds(_NS * _RPT, _RREM)])


@functools.lru_cache(maxsize=1)
def _make_sc_agg():
    return functools.partial(
        pl.kernel,
        out_type=(
            jax.ShapeDtypeStruct((_N, _D), jnp.float32),
            jax.ShapeDtypeStruct((_N, _D), jnp.float32),
        ),
        mesh=plsc.VectorSubcoreMesh(
            core_axis_name="c", subcore_axis_name="s",
            num_cores=_NC, num_subcores=_NS),
        compiler_params=pltpu.CompilerParams(needs_layout_passes=False),
        scratch_types=[
            pltpu.VMEM((_SUP,), jnp.int32),
            pltpu.VMEM((_SUP,), jnp.int32),
            pltpu.VMEM((_SUP,), jnp.float32),
            pltpu.VMEM((_CB, _D), jnp.float32),
            pltpu.VMEM((_CB, _D), jnp.float32),
            pltpu.VMEM((max(_TAIL, _L), _D), jnp.float32),
            pltpu.VMEM_SHARED((_N, _D), jnp.float32),
            pltpu.SemaphoreType.DMA,
            pltpu.SemaphoreType.DMA,
        ],
    )(_sc_agg_body)


def kernel(mean, var, edge_index, adj_w0, adj_w1, W_mean, b_mean, W_var,
           b_var):
    x2 = _dense_transform(mean, var, W_mean, b_mean, W_var, b_var)
    x2 = x2.reshape(2 * _N, _D)
    dst = edge_index[0]
    src = edge_index[1]
    w = jnp.concatenate([adj_w0, adj_w1])
    z = jnp.zeros((_N, _D), jnp.float32)
    m_out, v_out = _make_sc_agg()(x2, dst, src, w, z)
    return (m_out, v_out)


# speedup vs baseline: 1.0043x; 1.0043x over previous
import functools

import jax
import jax.numpy as jnp
from jax import lax
from jax.experimental import pallas as pl
from jax.experimental.pallas import tpu as pltpu
from jax.experimental.pallas import tpu_sc as plsc

_N = 10000
_E = 320000
_D = 128

_NC = 2
_NS = 16
_L = 16

_EPT = _E // _NS
_CB = 128
_SUP = 4096
_NSUP = -(-_EPT // _SUP)
_TAIL = _EPT - (_EPT // _CB) * _CB
_RPT = (_N // _NS) // 8 * 8
_RREM = _N - _RPT * _NS

_ROW_BLK = 1000


def _dense_body(mean_ref, var_ref, wm_ref, bm_ref, wv_ref, bv_ref, out_ref):
    m = jnp.dot(mean_ref[...], wm_ref[...],
                preferred_element_type=jnp.float32) + bm_ref[...]
    m = jnp.where(m > 0.0, m, jnp.exp(m) - 1.0)
    v = jnp.dot(var_ref[...], wv_ref[...],
                preferred_element_type=jnp.float32) + bv_ref[...]
    v = jnp.maximum(v, 0.0) + 1e-6
    att = jnp.exp(-v)
    out_ref[0] = m * att
    out_ref[1] = (v * att) * att


def _dense_transform(mean, var, W_mean, b_mean, W_var, b_var):
    return pl.pallas_call(
        _dense_body,
        grid=(_N // _ROW_BLK,),
        in_specs=[
            pl.BlockSpec((_ROW_BLK, _D), lambda i: (i, 0)),
            pl.BlockSpec((_ROW_BLK, _D), lambda i: (i, 0)),
            pl.BlockSpec((_D, _D), lambda i: (0, 0)),
            pl.BlockSpec((1, _D), lambda i: (0, 0)),
            pl.BlockSpec((_D, _D), lambda i: (0, 0)),
            pl.BlockSpec((1, _D), lambda i: (0, 0)),
        ],
        out_specs=pl.BlockSpec((2, _ROW_BLK, _D), lambda i: (0, i, 0)),
        out_shape=jax.ShapeDtypeStruct((2, _N, _D), jnp.float32),
    )(mean, var, W_mean, b_mean.reshape(1, _D), W_var, b_var.reshape(1, _D))


def _sc_agg_body(x_hbm, dst_hbm, src_hbm, w_hbm, z_hbm, m_out, v_out,
                 srcs, dsts, ws, rows0, rows1, rowst, acc, sem0, sem1):
    sid = lax.axis_index("s")
    cid = lax.axis_index("c")

    pltpu.sync_copy(z_hbm.at[pl.ds(sid * _RPT, _RPT)],
                    acc.at[pl.ds(sid * _RPT, _RPT)])

    @pl.when(sid == 0)
    def _():
        pltpu.sync_copy(z_hbm.at[pl.ds(_NS * _RPT, _RREM)],
                        acc.at[pl.ds(_NS * _RPT, _RREM)])

    plsc.subcore_barrier()

    ebase = sid * _EPT
    xoff = cid * _N
    woff = cid * _E

    def issue(ci, buf, sem):
        pltpu.async_copy(x_hbm.at[srcs.at[pl.ds(ci * _CB, _CB)]], buf, sem)

    def drain(buf, sem):
        pltpu.make_async_copy(x_hbm.at[pl.ds(0, _CB)], buf, sem).wait()

    def process(ci, buf):
        base = ci * _CB

        @plsc.parallel_loop(0, _CB, unroll=4)
        def _(e):
            wv = plsc.load_gather(ws, [jnp.full((_L,), base + e, jnp.int32)])
            for j in range(_D // _L):
                sl = pl.ds(j * _L, _L)
                buf[e, sl] = buf[e, sl] * wv

        pltpu.sync_copy(buf, acc.at[dsts.at[pl.ds(base, _CB)]], add=True)

    for s in range(_NSUP):
        slen = _SUP if s < _NSUP - 1 else _EPT - (_NSUP - 1) * _SUP
        nfull = slen // _CB
        stail = slen - nfull * _CB
        sb = ebase + s * _SUP

        pltpu.sync_copy(src_hbm.at[pl.ds(sb, slen)], srcs.at[pl.ds(0, slen)])
        pltpu.sync_copy(dst_hbm.at[pl.ds(sb, slen)], dsts.at[pl.ds(0, slen)])
        pltpu.sync_copy(w_hbm.at[pl.ds(woff + sb, slen)],
                        ws.at[pl.ds(0, slen)])

        @plsc.parallel_loop(0, slen // _L, unroll=4)
        def _(i):
            sl = pl.ds(i * _L, _L)
            srcs[sl] = srcs[sl] + xoff

        issue(0, rows0, sem0)

        def ring_body(i, carry, nfull=nfull):
            g = i * 2
            issue(g + 1, rows1, sem1)
            drain(rows0, sem0)
            process(g, rows0)

            @pl.when(g + 2 < nfull)
            def _():
                issue(g + 2, rows0, sem0)

            drain(rows1, sem1)
            process(g + 1, rows1)
            return carry

        lax.fori_loop(0, nfull // 2, ring_body, 0)

        if stail:
            tb = nfull * _CB
            pltpu.async_copy(
                x_hbm.at[srcs.at[pl.ds(tb, stail)]], rowst, sem0).wait()

            @plsc.parallel_loop(0, stail, unroll=4)
            def _(e):
                wv = plsc.load_gather(
                    ws, [jnp.full((_L,), tb + e, jnp.int32)])
                for j in range(_D // _L):
                    sl = pl.ds(j * _L, _L)
                    rowst[e, sl] = rowst[e, sl] * wv
            pltpu.sync_copy(rowst, acc.at[dsts.at[pl.ds(tb, stail)]],
                            add=True)

    plsc.subcore_barrier()

    @pl.when(cid == 0)
    def _():
        pltpu.sync_copy(acc.at[pl.ds(sid * _RPT, _RPT)],
                        m_out.at[pl.ds(sid * _RPT, _RPT)])

    @pl.when((cid == 0) & (sid == 0))
    def _():
        pltpu.sync_copy(acc.at[pl.ds(_NS * _RPT, _RREM)],
                        m_out.at[pl.ds(_NS * _RPT, _RREM)])

    @pl.when(cid == 1)
    def _():
        pltpu.sync_copy(acc.at[pl.ds(sid * _RPT, _RPT)],
                        v_out.at[pl.ds(sid * _RPT, _RPT)])

    @pl.when((cid == 1) & (sid == 0))
    def _():
        pltpu.sync_copy(acc.at[pl.ds(_NS * _RPT, _RREM)],
                        v_out.at[pl.ds(_NS * _RPT, _RREM)])


@functools.lru_cache(maxsize=1)
def _make_sc_agg():
    return functools.partial(
        pl.kernel,
        out_type=(
            jax.ShapeDtypeStruct((_N, _D), jnp.float32),
            jax.ShapeDtypeStruct((_N, _D), jnp.float32),
        ),
        mesh=plsc.VectorSubcoreMesh(
            core_axis_name="c", subcore_axis_name="s",
            num_cores=_NC, num_subcores=_NS),
        compiler_params=pltpu.CompilerParams(needs_layout_passes=False),
        scratch_types=[
            pltpu.VMEM((_SUP,), jnp.int32),
            pltpu.VMEM((_SUP,), jnp.int32),
            pltpu.VMEM((_SUP,), jnp.float32),
            pltpu.VMEM((_CB, _D), jnp.float32),
            pltpu.VMEM((_CB, _D), jnp.float32),
            pltpu.VMEM((max(_TAIL, _L), _D), jnp.float32),
            pltpu.VMEM_SHARED((_N, _D), jnp.float32),
            pltpu.SemaphoreType.DMA,
            pltpu.SemaphoreType.DMA,
        ],
    )(_sc_agg_body)


def kernel(mean, var, edge_index, adj_w0, adj_w1, W_mean, b_mean, W_var,
           b_var):
    x2 = _dense_transform(mean, var, W_mean, b_mean, W_var, b_var)
    x2 = x2.reshape(2 * _N, _D)
    dst = edge_index[0]
    src = edge_index[1]
    w = jnp.concatenate([adj_w0, adj_w1])
    z = jnp.zeros((_N, _D), jnp.float32)
    m_out, v_out = _make_sc_agg()(x2, dst, src, w, z)
    return (m_out, v_out)
